# Initial kernel scaffold; baseline (speedup 1.0000x reference)
import functools

import jax
import jax.numpy as jnp
from jax import lax
from jax.experimental import pallas as pl
from jax.experimental.pallas import tpu as pltpu
from jax.experimental.pallas import tpu_sc as plsc

N = 10000
E = 320000
D_IN = 128
D_OUT = 128
H = 8
DH = 16

NTILE = 32
TPC = E // NTILE
C = 80
NCHUNK = TPC // C
RPT = N // 16
ROW = 144


def _proj_body(h_ref, wq_ref, bq_ref, wkv_ref, bkv_ref, q_ref, kv_ref):
    h = h_ref[...]
    q = jnp.dot(h, wq_ref[...], preferred_element_type=jnp.float32) + bq_ref[...]
    q_ref[...] = q * (1.0 / (DH ** 0.5))
    kv_ref[...] = jnp.dot(h, wkv_ref[...], preferred_element_type=jnp.float32) + bkv_ref[...]


def _proj(h, Qw, Qb, Kw, Kb, Vw, Vb):
    wkv = jnp.concatenate([Kw, Vw], axis=1)
    bkv = jnp.concatenate([Kb, Vb])[None, :]
    bq = Qb[None, :]
    nb = 5
    rb = N // nb
    return pl.pallas_call(
        _proj_body,
        grid=(nb,),
        in_specs=[
            pl.BlockSpec((rb, D_IN), lambda i: (i, 0)),
            pl.BlockSpec((D_IN, D_OUT), lambda i: (0, 0)),
            pl.BlockSpec((1, D_OUT), lambda i: (0, 0)),
            pl.BlockSpec((D_IN, 2 * D_OUT), lambda i: (0, 0)),
            pl.BlockSpec((1, 2 * D_OUT), lambda i: (0, 0)),
        ],
        out_specs=[
            pl.BlockSpec((rb, D_OUT), lambda i: (i, 0)),
            pl.BlockSpec((rb, 2 * D_OUT), lambda i: (i, 0)),
        ],
        out_shape=[
            jax.ShapeDtypeStruct((N, D_OUT), jnp.float32),
            jax.ShapeDtypeStruct((N, 2 * D_OUT), jnp.float32),
        ],
    )(h, Qw, bq, wkv, bkv)


def _edge_body(kv_hbm, q_hbm, src_hbm, dst_hbm, zero_hbm, wv_out, z_out,
               src_v, dst_v, kv_buf, q_buf, msg_buf, s16, acc, sem1, sem2):
    cid = lax.axis_index("c")
    sid = lax.axis_index("s")
    pltpu.sync_copy(zero_hbm, acc.at[pl.ds(sid * RPT, RPT), :])
    plsc.subcore_barrier()

    mask8 = lax.iota(jnp.int32, 16) < 8
    base_t = (cid * 16 + sid) * TPC

    def chunk_body(i, carry):
        base = base_t + i * C
        pltpu.sync_copy(src_hbm.at[pl.ds(base, C)], src_v)
        pltpu.sync_copy(dst_hbm.at[pl.ds(base, C)], dst_v)
        cp1 = pltpu.async_copy(kv_hbm.at[src_v], kv_buf, sem1)
        cp2 = pltpu.async_copy(q_hbm.at[dst_v], q_buf, sem2)
        cp1.wait()
        cp2.wait()

        def edge_body(e, c2):
            for h in range(H):
                kq = kv_buf[e, pl.ds(16 * h, 16)] * q_buf[e, pl.ds(16 * h, 16)]
                s16[h] = jnp.sum(kq)
            sv = s16[...]
            sv = jnp.exp(jnp.clip(sv, -5.0, 5.0))
            sv = jnp.where(mask8, sv, 0.0)
            msg_buf[e, pl.ds(128, 16)] = sv
            s16[...] = sv
            for h in range(H):
                s = s16[h]
                msg_buf[e, pl.ds(16 * h, 16)] = kv_buf[e, pl.ds(128 + 16 * h, 16)] * s
            return c2

        lax.fori_loop(0, C, edge_body, 0)
        pltpu.sync_copy(msg_buf, acc.at[dst_v], add=True)
        return carry

    lax.fori_loop(0, NCHUNK, chunk_body, 0)
    plsc.subcore_barrier()
    r0 = sid * RPT
    pltpu.sync_copy(acc.at[pl.ds(r0, RPT), pl.ds(0, 128)],
                    wv_out.at[cid, pl.ds(r0, RPT), :])
    pltpu.sync_copy(acc.at[pl.ds(r0, RPT), pl.ds(128, 16)],
                    z_out.at[cid, pl.ds(r0, RPT), :])


def _edge_phase(kv, q, src, dst):
    zero = jnp.zeros((RPT, ROW), dtype=jnp.float32)
    mesh = plsc.VectorSubcoreMesh(core_axis_name="c", subcore_axis_name="s")
    f = pl.kernel(
        _edge_body,
        out_type=(
            jax.ShapeDtypeStruct((2, N, 128), jnp.float32),
            jax.ShapeDtypeStruct((2, N, 16), jnp.float32),
        ),
        mesh=mesh,
        scratch_types=[
            pltpu.VMEM((C,), jnp.int32),
            pltpu.VMEM((C,), jnp.int32),
            pltpu.VMEM((C, 256), jnp.float32),
            pltpu.VMEM((C, 128), jnp.float32),
            pltpu.VMEM((C, ROW), jnp.float32),
            pltpu.VMEM((16,), jnp.float32),
            pltpu.VMEM_SHARED((N, ROW), jnp.float32),
            pltpu.SemaphoreType.DMA,
            pltpu.SemaphoreType.DMA,
        ],
    )
    return f(kv, q, src, dst, zero)


def _post_a_body(wv_ref, z_ref, h_ref, x_ref, st_ref):
    i = pl.program_id(0)
    wv = wv_ref[0] + wv_ref[1]
    z16 = z_ref[0] + z_ref[1]
    rec = jnp.where(z16 > 0.0, 1.0 / jnp.where(z16 > 0.0, z16, 1.0), 0.0)
    rows = lax.broadcasted_iota(jnp.int32, (16, 128), 0)
    cols = lax.broadcasted_iota(jnp.int32, (16, 128), 1)
    expand = jnp.where(cols // 16 == rows, 1.0, 0.0)
    attn = wv * jnp.dot(rec, expand, preferred_element_type=jnp.float32)
    x = h_ref[...] + attn
    x_ref[...] = x

    @pl.when(i == 0)
    def _():
        st_ref[...] = jnp.zeros_like(st_ref)

    st_ref[0, :] += jnp.sum(x, axis=0)
    st_ref[1, :] += jnp.sum(x * x, axis=0)


def _post_b_body(x_ref, st_ref, g1_ref, be1_ref, w1_ref, b1_ref, w2_ref,
                 b2_ref, y_ref, st2_ref):
    i = pl.program_id(0)
    mean = st_ref[0, :] * (1.0 / N)
    var = st_ref[1, :] * (1.0 / N) - mean * mean
    scale = lax.rsqrt(var + 1e-5) * g1_ref[0, :]
    xn = (x_ref[...] - mean) * scale + be1_ref[0, :]
    f = jnp.dot(jax.nn.relu(
        jnp.dot(xn, w1_ref[...], preferred_element_type=jnp.float32) + b1_ref[...]),
        w2_ref[...], preferred_element_type=jnp.float32) + b2_ref[...]
    y = xn + f
    y_ref[...] = y

    @pl.when(i == 0)
    def _():
        st2_ref[...] = jnp.zeros_like(st2_ref)

    st2_ref[0, :] += jnp.sum(y, axis=0)
    st2_ref[1, :] += jnp.sum(y * y, axis=0)


def _post_c_body(y_ref, st_ref, g2_ref, be2_ref, o_ref):
    mean = st_ref[0, :] * (1.0 / N)
    var = st_ref[1, :] * (1.0 / N) - mean * mean
    scale = lax.rsqrt(var + 1e-5) * g2_ref[0, :]
    o_ref[...] = (y_ref[...] - mean) * scale + be2_ref[0, :]


def _post(wv, z, h, gamma1, beta1, W1, b1, W2, b2, gamma2, beta2):
    nb = 5
    rb = N // nb
    full = lambda s: pl.BlockSpec(s, lambda i, _s=s: tuple(0 for _ in _s))
    x, st1 = pl.pallas_call(
        _post_a_body,
        grid=(nb,),
        in_specs=[
            pl.BlockSpec((2, rb, 128), lambda i: (0, i, 0)),
            pl.BlockSpec((2, rb, 16), lambda i: (0, i, 0)),
            pl.BlockSpec((rb, 128), lambda i: (i, 0)),
        ],
        out_specs=[
            pl.BlockSpec((rb, 128), lambda i: (i, 0)),
            pl.BlockSpec((2, 128), lambda i: (0, 0)),
        ],
        out_shape=[
            jax.ShapeDtypeStruct((N, 128), jnp.float32),
            jax.ShapeDtypeStruct((2, 128), jnp.float32),
        ],
    )(wv, z, h)

    y, st2 = pl.pallas_call(
        _post_b_body,
        grid=(nb,),
        in_specs=[
            pl.BlockSpec((rb, 128), lambda i: (i, 0)),
            full((2, 128)),
            full((1, 128)),
            full((1, 128)),
            full((128, 256)),
            full((1, 256)),
            full((256, 128)),
            full((1, 128)),
        ],
        out_specs=[
            pl.BlockSpec((rb, 128), lambda i: (i, 0)),
            pl.BlockSpec((2, 128), lambda i: (0, 0)),
        ],
        out_shape=[
            jax.ShapeDtypeStruct((N, 128), jnp.float32),
            jax.ShapeDtypeStruct((2, 128), jnp.float32),
        ],
    )(x, st1, gamma1[None, :], beta1[None, :], W1, b1[None, :], W2, b2[None, :])

    out = pl.pallas_call(
        _post_c_body,
        grid=(nb,),
        in_specs=[
            pl.BlockSpec((rb, 128), lambda i: (i, 0)),
            full((2, 128)),
            full((1, 128)),
            full((1, 128)),
        ],
        out_specs=pl.BlockSpec((rb, 128), lambda i: (i, 0)),
        out_shape=jax.ShapeDtypeStruct((N, 128), jnp.float32),
    )(y, st2, gamma2[None, :], beta2[None, :])
    return out


@jax.jit
def kernel(h, edge_index, Qw, Qb, Kw, Kb, Vw, Vb, gamma1, beta1, W1, b1, W2,
           b2, gamma2, beta2):
    q, kv = _proj(h, Qw, Qb, Kw, Kb, Vw, Vb)
    src = edge_index[0]
    dst = edge_index[1]
    wv, z = _edge_phase(kv, q, src, dst)
    return _post(wv, z, h, gamma1, beta1, W1, b1, W2, b2, gamma2, beta2)


# speedup vs baseline: 21.9490x; 21.9490x over previous
import functools

import jax
import jax.numpy as jnp
from jax import lax
from jax.experimental import pallas as pl
from jax.experimental.pallas import tpu as pltpu
from jax.experimental.pallas import tpu_sc as plsc

N = 10000
E = 320000
D_IN = 128
D_OUT = 128
H = 8
DH = 16

NTILE = 32
TPC = E // NTILE
C = 80
NCHUNK = TPC // C
NP = 10240
RPT = NP // 16
ROW = 136


def _proj_body(h_ref, wq_ref, bq_ref, wkv_ref, bkv_ref, q_ref, kv_ref):
    h = h_ref[...]
    q = jnp.dot(h, wq_ref[...], preferred_element_type=jnp.float32) + bq_ref[...]
    q_ref[...] = q * (1.0 / (DH ** 0.5))
    kv_ref[...] = jnp.dot(h, wkv_ref[...], preferred_element_type=jnp.float32) + bkv_ref[...]


def _proj(h, Qw, Qb, Kw, Kb, Vw, Vb):
    wkv = jnp.concatenate([Kw, Vw], axis=1)
    bkv = jnp.concatenate([Kb, Vb])[None, :]
    bq = Qb[None, :]
    nb = 5
    rb = N // nb
    return pl.pallas_call(
        _proj_body,
        grid=(nb,),
        in_specs=[
            pl.BlockSpec((rb, D_IN), lambda i: (i, 0)),
            pl.BlockSpec((D_IN, D_OUT), lambda i: (0, 0)),
            pl.BlockSpec((1, D_OUT), lambda i: (0, 0)),
            pl.BlockSpec((D_IN, 2 * D_OUT), lambda i: (0, 0)),
            pl.BlockSpec((1, 2 * D_OUT), lambda i: (0, 0)),
        ],
        out_specs=[
            pl.BlockSpec((rb, D_OUT), lambda i: (i, 0)),
            pl.BlockSpec((rb, 2 * D_OUT), lambda i: (i, 0)),
        ],
        out_shape=[
            jax.ShapeDtypeStruct((N, D_OUT), jnp.float32),
            jax.ShapeDtypeStruct((N, 2 * D_OUT), jnp.float32),
        ],
    )(h, Qw, bq, wkv, bkv)


def _edge_body(kv_hbm, q_hbm, src_hbm, dst_hbm, zero_hbm, wv_out, z_out,
               src_v, dst_v, kv_buf, q_buf, msg_buf, s16, acc, sem1, sem2):
    cid = lax.axis_index("c")
    sid = lax.axis_index("s")
    pltpu.sync_copy(zero_hbm, acc.at[pl.ds(sid * RPT, RPT), :])
    plsc.subcore_barrier()

    lanes = lax.iota(jnp.int32, 16)
    mask8 = lanes < 8
    mask15 = lanes == 15
    col128 = lanes + 128
    hvecs = [jnp.full((16,), h, jnp.int32) for h in range(H)]
    base_t = (cid * 16 + sid) * TPC

    def chunk_body(i, carry):
        base = base_t + i * C
        pltpu.sync_copy(src_hbm.at[pl.ds(base, C)], src_v)
        pltpu.sync_copy(dst_hbm.at[pl.ds(base, C)], dst_v)
        cp1 = pltpu.async_copy(kv_hbm.at[src_v], kv_buf, sem1)
        cp2 = pltpu.async_copy(q_hbm.at[dst_v], q_buf, sem2)
        cp1.wait()
        cp2.wait()

        def edge_body(e, c2):
            for h in range(H):
                kq = kv_buf[e, pl.ds(16 * h, 16)] * q_buf[e, pl.ds(16 * h, 16)]
                plsc.store_scatter(s16, [hvecs[h]], plsc.cumsum(kq), mask=mask15)
            sv = s16[...]
            sv = jnp.exp(jnp.clip(sv, -5.0, 5.0))
            sv = jnp.where(mask8, sv, 0.0)
            plsc.store_scatter(msg_buf, [jnp.broadcast_to(e, (16,)), col128],
                               sv, mask=mask8)
            for h in range(H):
                msg_buf[e, pl.ds(16 * h, 16)] = (
                    kv_buf[e, pl.ds(128 + 16 * h, 16)] * sv[h])
            return c2

        lax.fori_loop(0, C, edge_body, 0)
        pltpu.sync_copy(msg_buf, acc.at[dst_v], add=True)
        return carry

    lax.fori_loop(0, NCHUNK, chunk_body, 0)
    plsc.subcore_barrier()
    r0 = sid * RPT
    pltpu.sync_copy(acc.at[pl.ds(r0, RPT), pl.ds(0, 128)],
                    wv_out.at[cid, pl.ds(r0, RPT), :])
    pltpu.sync_copy(acc.at[pl.ds(r0, RPT), pl.ds(128, 8)],
                    z_out.at[cid, pl.ds(r0, RPT), :])


def _edge_phase(kv, q, src, dst):
    zero = jnp.zeros((RPT, ROW), dtype=jnp.float32)
    mesh = plsc.VectorSubcoreMesh(core_axis_name="c", subcore_axis_name="s")
    f = pl.kernel(
        _edge_body,
        out_type=(
            jax.ShapeDtypeStruct((2, NP, 128), jnp.float32),
            jax.ShapeDtypeStruct((2, NP, 8), jnp.float32),
        ),
        mesh=mesh,
        compiler_params=pltpu.CompilerParams(
            needs_layout_passes=False, use_tc_tiling_on_sc=False),
        scratch_types=[
            pltpu.VMEM((C,), jnp.int32),
            pltpu.VMEM((C,), jnp.int32),
            pltpu.VMEM((C, 256), jnp.float32),
            pltpu.VMEM((C, 128), jnp.float32),
            pltpu.VMEM((C, ROW), jnp.float32),
            pltpu.VMEM((16,), jnp.float32),
            pltpu.VMEM_SHARED((NP, ROW), jnp.float32),
            pltpu.SemaphoreType.DMA,
            pltpu.SemaphoreType.DMA,
        ],
    )
    return f(kv, q, src, dst, zero)


def _post_a_body(wv_ref, z_ref, h_ref, x_ref, st_ref):
    i = pl.program_id(0)
    wv = wv_ref[0] + wv_ref[1]
    z8 = z_ref[0] + z_ref[1]
    rec = jnp.where(z8 > 0.0, 1.0 / jnp.where(z8 > 0.0, z8, 1.0), 0.0)
    rows = lax.broadcasted_iota(jnp.int32, (8, 128), 0)
    cols = lax.broadcasted_iota(jnp.int32, (8, 128), 1)
    expand = jnp.where(cols // 16 == rows, 1.0, 0.0)
    attn = wv * jnp.dot(rec, expand, preferred_element_type=jnp.float32)
    x = h_ref[...] + attn
    x_ref[...] = x

    @pl.when(i == 0)
    def _():
        st_ref[...] = jnp.zeros_like(st_ref)

    st_ref[0, :] += jnp.sum(x, axis=0)
    st_ref[1, :] += jnp.sum(x * x, axis=0)


def _post_b_body(x_ref, st_ref, g1_ref, be1_ref, w1_ref, b1_ref, w2_ref,
                 b2_ref, y_ref, st2_ref):
    i = pl.program_id(0)
    mean = st_ref[0, :] * (1.0 / N)
    var = st_ref[1, :] * (1.0 / N) - mean * mean
    scale = lax.rsqrt(var + 1e-5) * g1_ref[0, :]
    xn = (x_ref[...] - mean) * scale + be1_ref[0, :]
    f = jnp.dot(jax.nn.relu(
        jnp.dot(xn, w1_ref[...], preferred_element_type=jnp.float32) + b1_ref[...]),
        w2_ref[...], preferred_element_type=jnp.float32) + b2_ref[...]
    y = xn + f
    y_ref[...] = y

    @pl.when(i == 0)
    def _():
        st2_ref[...] = jnp.zeros_like(st2_ref)

    st2_ref[0, :] += jnp.sum(y, axis=0)
    st2_ref[1, :] += jnp.sum(y * y, axis=0)


def _post_c_body(y_ref, st_ref, g2_ref, be2_ref, o_ref):
    mean = st_ref[0, :] * (1.0 / N)
    var = st_ref[1, :] * (1.0 / N) - mean * mean
    scale = lax.rsqrt(var + 1e-5) * g2_ref[0, :]
    o_ref[...] = (y_ref[...] - mean) * scale + be2_ref[0, :]


def _post(wv, z, h, gamma1, beta1, W1, b1, W2, b2, gamma2, beta2):
    nb = 5
    rb = N // nb
    full = lambda s: pl.BlockSpec(s, lambda i, _s=s: tuple(0 for _ in _s))
    x, st1 = pl.pallas_call(
        _post_a_body,
        grid=(nb,),
        in_specs=[
            pl.BlockSpec((2, rb, 128), lambda i: (0, i, 0)),
            pl.BlockSpec((2, rb, 8), lambda i: (0, i, 0)),
            pl.BlockSpec((rb, 128), lambda i: (i, 0)),
        ],
        out_specs=[
            pl.BlockSpec((rb, 128), lambda i: (i, 0)),
            pl.BlockSpec((2, 128), lambda i: (0, 0)),
        ],
        out_shape=[
            jax.ShapeDtypeStruct((N, 128), jnp.float32),
            jax.ShapeDtypeStruct((2, 128), jnp.float32),
        ],
    )(wv, z, h)

    y, st2 = pl.pallas_call(
        _post_b_body,
        grid=(nb,),
        in_specs=[
            pl.BlockSpec((rb, 128), lambda i: (i, 0)),
            full((2, 128)),
            full((1, 128)),
            full((1, 128)),
            full((128, 256)),
            full((1, 256)),
            full((256, 128)),
            full((1, 128)),
        ],
        out_specs=[
            pl.BlockSpec((rb, 128), lambda i: (i, 0)),
            pl.BlockSpec((2, 128), lambda i: (0, 0)),
        ],
        out_shape=[
            jax.ShapeDtypeStruct((N, 128), jnp.float32),
            jax.ShapeDtypeStruct((2, 128), jnp.float32),
        ],
    )(x, st1, gamma1[None, :], beta1[None, :], W1, b1[None, :], W2, b2[None, :])

    out = pl.pallas_call(
        _post_c_body,
        grid=(nb,),
        in_specs=[
            pl.BlockSpec((rb, 128), lambda i: (i, 0)),
            full((2, 128)),
            full((1, 128)),
            full((1, 128)),
        ],
        out_specs=pl.BlockSpec((rb, 128), lambda i: (i, 0)),
        out_shape=jax.ShapeDtypeStruct((N, 128), jnp.float32),
    )(y, st2, gamma2[None, :], beta2[None, :])
    return out


@jax.jit
def kernel(h, edge_index, Qw, Qb, Kw, Kb, Vw, Vb, gamma1, beta1, W1, b1, W2,
           b2, gamma2, beta2):
    q, kv = _proj(h, Qw, Qb, Kw, Kb, Vw, Vb)
    src = edge_index[0]
    dst = edge_index[1]
    wv, z = _edge_phase(kv, q, src, dst)
    return _post(wv, z, h, gamma1, beta1, W1, b1, W2, b2, gamma2, beta2)
